# Initial kernel scaffold; baseline (speedup 1.0000x reference)
import jax
import jax.numpy as jnp
from jax.experimental import pallas as pl


def kernel(x, adj_matrix, W, a_src, a_dst, bias):
    raise NotImplementedError("write your pallas kernel here")


# speedup vs baseline: 1.3164x; 1.3164x over previous
import functools

import jax
import jax.numpy as jnp
from jax.experimental import pallas as pl


def _gat_kernel(x_ref, adj_ref, w_ref, asrc_ref, adst_ref, bias_ref, out_ref):
    N = adj_ref.shape[1]
    H = w_ref.shape[1]

    x = x_ref[0]
    xp = jax.lax.dot(x, w_ref[...], preferred_element_type=jnp.float32)

    s_src = jax.lax.dot_general(
        xp, asrc_ref[...], (((1,), (1,)), ((), ())),
        preferred_element_type=jnp.float32)
    s_dst = jax.lax.dot_general(
        adst_ref[...], xp, (((1,), (1,)), ((), ())),
        preferred_element_type=jnp.float32)

    logits = s_src + s_dst
    logits = jnp.where(logits >= 0, logits, 0.2 * logits)

    row = jax.lax.broadcasted_iota(jnp.int32, (N, N), 0)
    col = jax.lax.broadcasted_iota(jnp.int32, (N, N), 1)
    mask = (adj_ref[0] != 0) | (row == col)
    logits = jnp.where(mask, logits, -1e9)

    m = jnp.max(logits, axis=0, keepdims=True)
    e = jnp.exp(logits - m)

    ones = jnp.ones((xp.shape[0], 1), dtype=jnp.float32)
    xp_aug = jnp.concatenate([xp, ones], axis=1)
    num_aug = jax.lax.dot_general(
        e, xp_aug, (((0,), (0,)), ((), ())),
        preferred_element_type=jnp.float32)

    denom = num_aug[:, H:H + 1] + 1e-16
    out_ref[0] = num_aug[:, :H] / denom + bias_ref[...]


@jax.jit
def kernel(x, adj_matrix, W, a_src, a_dst, bias):
    B, S, N, F = x.shape
    H = W.shape[1]
    T = B * S

    xf = x.reshape(T, N, F)
    adjf = adj_matrix.reshape(T, N, N)
    a_src2 = a_src.reshape(1, H)
    a_dst2 = a_dst.reshape(1, H)
    bias2 = bias.reshape(1, H)

    out = pl.pallas_call(
        _gat_kernel,
        grid=(T,),
        in_specs=[
            pl.BlockSpec((1, N, F), lambda t: (t, 0, 0)),
            pl.BlockSpec((1, N, N), lambda t: (t, 0, 0)),
            pl.BlockSpec((F, H), lambda t: (0, 0)),
            pl.BlockSpec((1, H), lambda t: (0, 0)),
            pl.BlockSpec((1, H), lambda t: (0, 0)),
            pl.BlockSpec((1, H), lambda t: (0, 0)),
        ],
        out_specs=pl.BlockSpec((1, N, H), lambda t: (t, 0, 0)),
        out_shape=jax.ShapeDtypeStruct((T, N, H), jnp.float32),
    )(xf, adjf, W, a_src2, a_dst2, bias2)

    return out.reshape(B, S, N, H)


# speedup vs baseline: 1.3596x; 1.0328x over previous
import functools

import jax
import jax.numpy as jnp
from jax.experimental import pallas as pl


def _gat_kernel(x_ref, adj_ref, eye_ref, w_ref, asrc_ref, adst_ref, bias_ref,
                out_ref):
    N = adj_ref.shape[1]
    H = w_ref.shape[1]

    x = x_ref[0]
    xp = jax.lax.dot(x, w_ref[...], preferred_element_type=jnp.float32)

    s_src = jax.lax.dot_general(
        xp, asrc_ref[...], (((1,), (1,)), ((), ())),
        preferred_element_type=jnp.float32)
    s_dst = jax.lax.dot_general(
        adst_ref[...], xp, (((1,), (1,)), ((), ())),
        preferred_element_type=jnp.float32)

    s_max = jnp.max(s_src)
    mrow = s_max + s_dst
    mrow = jnp.maximum(mrow, 0.2 * mrow)

    logits = s_src + s_dst
    logits = jnp.maximum(logits, 0.2 * logits)

    mask = (adj_ref[0] + eye_ref[...]) != 0
    e = jnp.where(mask, jnp.exp2(logits - mrow), 0.0)

    ones = jnp.ones((xp.shape[0], 1), dtype=jnp.float32)
    xp_aug = jnp.concatenate([xp, ones], axis=1)
    num_aug = jax.lax.dot_general(
        e.astype(jnp.bfloat16), xp_aug.astype(jnp.bfloat16),
        (((0,), (0,)), ((), ())),
        preferred_element_type=jnp.float32)

    denom = num_aug[:, H:H + 1] + 1e-16
    out_ref[0] = num_aug[:, :H] / denom + bias_ref[...]


@jax.jit
def kernel(x, adj_matrix, W, a_src, a_dst, bias):
    B, S, N, F = x.shape
    H = W.shape[1]
    T = B * S

    xf = x.reshape(T, N, F)
    adjf = adj_matrix.reshape(T, N, N)
    log2e = jnp.float32(1.4426950408889634)
    a_src2 = (a_src * log2e).reshape(1, H)
    a_dst2 = (a_dst * log2e).reshape(1, H)
    bias2 = bias.reshape(1, H)
    eye = jnp.eye(N, dtype=jnp.float32)

    out = pl.pallas_call(
        _gat_kernel,
        grid=(T,),
        in_specs=[
            pl.BlockSpec((1, N, F), lambda t: (t, 0, 0)),
            pl.BlockSpec((1, N, N), lambda t: (t, 0, 0)),
            pl.BlockSpec((N, N), lambda t: (0, 0)),
            pl.BlockSpec((F, H), lambda t: (0, 0)),
            pl.BlockSpec((1, H), lambda t: (0, 0)),
            pl.BlockSpec((1, H), lambda t: (0, 0)),
            pl.BlockSpec((1, H), lambda t: (0, 0)),
        ],
        out_specs=pl.BlockSpec((1, N, H), lambda t: (t, 0, 0)),
        out_shape=jax.ShapeDtypeStruct((T, N, H), jnp.float32),
    )(xf, adjf, eye, W, a_src2, a_dst2, bias2)

    return out.reshape(B, S, N, H)
